# baseline (speedup 1.0000x reference)
import jax
import jax.numpy as jnp
from jax import lax
from jax.experimental import pallas as pl
from jax.experimental.pallas import tpu as pltpu
from jax.experimental.pallas import tpu_sc as plsc

W = 8
K = 16
F = 224
T = 224
FO = F - W + 1
BC = 4 * 192
NEG = float("-inf")
GCH = 8
FP = 224
HR = BC * FP

NC = 2
NS = 16
NW = NC * NS
NCHUNK = 2
CBC = BC // NCHUNK
SPW = CBC // NW
LANES = 16
NG = T // LANES
OUTP = 128


def _pool_body(x_ref, ol_ref, or_ref, rm_ref):
    a = x_ref[...]
    b = a[:, :-1, :] + a[:, 1:, :]
    b = b[:, :-2, :] + b[:, 2:, :]
    v = b[:, :-4, :] + b[:, 4:, :]
    h = v
    for d in (1, 2, 4):
        h = h + jnp.concatenate([h[:, :, d:], h[:, :, :d]], axis=2)
    col = lax.broadcasted_iota(jnp.int32, h.shape, 2)
    o = jnp.where(col < FO, h * (1.0 / (W * W)), NEG)
    rm = jnp.max(o, axis=2)
    rm_ref[...] = jnp.concatenate(
        [rm, jnp.full((GCH, FP - FO), NEG, jnp.float32)], axis=1)[:, None, :]
    opad = jnp.concatenate(
        [o, jnp.full((GCH, FP - FO, T), NEG, jnp.float32)], axis=1)
    ol_ref[...] = opad[:, :, 0:128].reshape(GCH * FP, 128)
    orr = jnp.concatenate(
        [opad[:, :, 128:T], jnp.full((GCH, FP, 256 - T), NEG, jnp.float32)],
        axis=2)
    or_ref[...] = orr.reshape(GCH * FP, 128)


def _tc_pool(xr):
    n = xr.shape[0]
    return pl.pallas_call(
        _pool_body,
        grid=(n // GCH,),
        in_specs=[pl.BlockSpec((GCH, F, T), lambda i: (i, 0, 0))],
        out_specs=[pl.BlockSpec((GCH * FP, 128), lambda i: (i, 0)),
                   pl.BlockSpec((GCH * FP, 128), lambda i: (i, 0)),
                   pl.BlockSpec((GCH, 1, T), lambda i: (i, 0, 0))],
        out_shape=[jax.ShapeDtypeStruct((n * FP, 128), jnp.float32),
                   jax.ShapeDtypeStruct((n * FP, 128), jnp.float32),
                   jax.ShapeDtypeStruct((n, 1, T), jnp.float32)],
    )(xr)


def _bmerge(a, b):
    return jnp.sort(jnp.maximum(a, jnp.flip(b)))


def _tournament(vs):
    level = [jnp.sort(v) for v in vs]
    while len(level) > 1:
        nxt = [_bmerge(level[i], level[i + 1])
               for i in range(0, len(level) - 1, 2)]
        if len(level) % 2:
            nxt.append(level[-1])
        level = nxt
    return level[0]


def _process_slice(bufl, bufr, rmb):
    lanes = lax.broadcasted_iota(jnp.int32, (LANES,), 0)
    srm = _tournament([rmb[pl.ds(g * LANES, LANES)] for g in range(NG)])
    smin = jnp.full((LANES,), jnp.min(srm), jnp.float32)
    s0 = jnp.full((LANES,), NEG, jnp.float32)

    def group(g, carry):
        S, smin = carry
        rm = rmb[pl.ds(g * LANES, LANES)]
        mask = rm >= smin

        def cond(c):
            m, _, _ = c
            return jnp.any(m)

        def step(c):
            m, S, smin = c
            ffsv = plsc.all_reduce_ffs(m)
            row = g * LANES + ffsv
            vs = [plsc.load_gather(bufl, [row, j * LANES + lanes])
                  for j in range(8)]
            vs += [plsc.load_gather(bufr, [row, j * LANES + lanes])
                   for j in range(NG - 8)]
            S = _bmerge(S, _tournament(vs))
            smin = jnp.full((LANES,), jnp.min(S), jnp.float32)
            m = jnp.logical_and(m, lanes != ffsv)
            m = jnp.logical_and(m, rm >= smin)
            return m, S, smin

        _, S, smin = lax.while_loop(cond, step, (mask, S, smin))
        return S, smin

    S, _ = lax.fori_loop(0, NG, group, (s0, smin))
    return jnp.sum(S) * (1.0 / K)


def _sc_topk_body(flatl, flatr, rowmax, out,
                  bl0, bl1, br0, br1, rmb0, rmb1, res,
                  seml0, seml1, semr0, semr1, semm0, semm1):
    w = lax.axis_index("s") * NC + lax.axis_index("c")
    base = w * SPW
    bls, brs, rmbs = (bl0, bl1), (br0, br1), (rmb0, rmb1)
    semls, semrs, semms = (seml0, seml1), (semr0, semr1), (semm0, semm1)
    zero = jnp.zeros((LANES,), jnp.float32)
    lanes = lax.broadcasted_iota(jnp.int32, (LANES,), 0)

    def start(sl, b):
        pltpu.async_copy(flatl.at[pl.ds((base + sl) * FP, FP)], bls[b], semls[b])
        pltpu.async_copy(flatr.at[pl.ds((base + sl) * FP, FP)], brs[b], semrs[b])
        pltpu.async_copy(rowmax.at[base + sl], rmbs[b], semms[b])

    for b in (0, 1):
        start(b, b)

    def pair(i, carry):
        r0 = carry
        for b in (0, 1):
            sl = i + b
            pltpu.make_async_copy(flatl.at[pl.ds(0, FP)], bls[b], semls[b]).wait()
            pltpu.make_async_copy(flatr.at[pl.ds(0, FP)], brs[b], semrs[b]).wait()
            pltpu.make_async_copy(rowmax.at[base], rmbs[b], semms[b]).wait()
            t = _process_slice(bls[b], brs[b], rmbs[b])

            @pl.when(sl + 2 < SPW)
            def _():
                start(sl + 2, b)

            r0 = jnp.where(lanes == sl, t, r0)
        return r0

    r0 = pl.loop(0, SPW, step=2, init_carry=zero)(pair)
    for j in range(OUTP // LANES):
        res[pl.ds(j * LANES, LANES)] = r0 if j == 0 else zero
    pltpu.sync_copy(res.at[pl.ds(0, OUTP)], out.at[w])


_sc_topk = pl.kernel(
    _sc_topk_body,
    out_type=jax.ShapeDtypeStruct((NW, OUTP), jnp.float32),
    mesh=plsc.VectorSubcoreMesh(core_axis_name="c", subcore_axis_name="s",
                                num_cores=NC, num_subcores=NS),
    scratch_types=[
        pltpu.VMEM((FP, 128), jnp.float32),
        pltpu.VMEM((FP, 128), jnp.float32),
        pltpu.VMEM((FP, 128), jnp.float32),
        pltpu.VMEM((FP, 128), jnp.float32),
        pltpu.VMEM((T,), jnp.float32),
        pltpu.VMEM((T,), jnp.float32),
        pltpu.VMEM((OUTP,), jnp.float32),
        pltpu.SemaphoreType.DMA,
        pltpu.SemaphoreType.DMA,
        pltpu.SemaphoreType.DMA,
        pltpu.SemaphoreType.DMA,
        pltpu.SemaphoreType.DMA,
        pltpu.SemaphoreType.DMA,
    ],
    compiler_params=pltpu.CompilerParams(needs_layout_passes=False),
)


def kernel(x):
    B, C, _, _ = x.shape
    xr = x.reshape(B * C, F, T)
    outs = []
    for k in range(NCHUNK):
        xc = jax.lax.slice_in_dim(xr, k * CBC, (k + 1) * CBC, axis=0)
        pooledl, pooledr, rowmax = _tc_pool(xc)
        o = _sc_topk(pooledl, pooledr, rowmax.reshape(CBC, T))
        outs.append(o[:, :SPW].reshape(CBC))
    return jnp.concatenate(outs).reshape(B, C)


# speedup vs baseline: 1.2778x; 1.2778x over previous
import jax
import jax.numpy as jnp
from jax import lax
from jax.experimental import pallas as pl
from jax.experimental.pallas import tpu as pltpu
from jax.experimental.pallas import tpu_sc as plsc

W = 8
K = 16
F = 224
T = 224
FO = F - W + 1
BC = 4 * 192
NEG = float("-inf")
GCH = 8
FP = 224
HR = BC * FP

NC = 2
NS = 16
NW = NC * NS
NCHUNK = 1
CBC = BC // NCHUNK
SPW = CBC // NW
LANES = 16
NG = T // LANES
OUTP = 128


def _pool_body(x_ref, ol_ref, or_ref, rm_ref):
    a = x_ref[...]
    b = a[:, :-1, :] + a[:, 1:, :]
    b = b[:, :-2, :] + b[:, 2:, :]
    v = b[:, :-4, :] + b[:, 4:, :]
    h = v
    for d in (1, 2, 4):
        h = h + jnp.concatenate([h[:, :, d:], h[:, :, :d]], axis=2)
    col = lax.broadcasted_iota(jnp.int32, h.shape, 2)
    o = jnp.where(col < FO, h * (1.0 / (W * W)), NEG)
    rm = jnp.max(o, axis=2)
    rm_ref[...] = jnp.concatenate(
        [rm, jnp.full((GCH, FP - FO), NEG, jnp.float32)], axis=1)[:, None, :]
    opad = jnp.concatenate(
        [o, jnp.full((GCH, FP - FO, T), NEG, jnp.float32)], axis=1)
    ol_ref[...] = opad[:, :, 0:128].reshape(GCH * FP, 128)
    orr = jnp.concatenate(
        [opad[:, :, 128:T], jnp.full((GCH, FP, 256 - T), NEG, jnp.float32)],
        axis=2)
    or_ref[...] = orr.reshape(GCH * FP, 128)


def _tc_pool(xr):
    n = xr.shape[0]
    return pl.pallas_call(
        _pool_body,
        grid=(n // GCH,),
        in_specs=[pl.BlockSpec((GCH, F, T), lambda i: (i, 0, 0))],
        out_specs=[pl.BlockSpec((GCH * FP, 128), lambda i: (i, 0)),
                   pl.BlockSpec((GCH * FP, 128), lambda i: (i, 0)),
                   pl.BlockSpec((GCH, 1, T), lambda i: (i, 0, 0))],
        out_shape=[jax.ShapeDtypeStruct((n * FP, 128), jnp.float32),
                   jax.ShapeDtypeStruct((n * FP, 128), jnp.float32),
                   jax.ShapeDtypeStruct((n, 1, T), jnp.float32)],
    )(xr)


def _bmerge(a, b):
    return jnp.sort(jnp.maximum(a, jnp.flip(b)))


def _tournament(vs):
    level = [jnp.sort(v) for v in vs]
    while len(level) > 1:
        nxt = [_bmerge(level[i], level[i + 1])
               for i in range(0, len(level) - 1, 2)]
        if len(level) % 2:
            nxt.append(level[-1])
        level = nxt
    return level[0]


def _process_slice(bufl, bufr, rmb):
    lanes = lax.broadcasted_iota(jnp.int32, (LANES,), 0)
    srm = _tournament([rmb[pl.ds(g * LANES, LANES)] for g in range(NG)])
    smin = jnp.full((LANES,), jnp.min(srm), jnp.float32)
    s0 = jnp.full((LANES,), NEG, jnp.float32)

    def group(g, carry):
        S, smin = carry
        rm = rmb[pl.ds(g * LANES, LANES)]
        mask = rm >= smin

        def cond(c):
            m, _, _ = c
            return jnp.any(m)

        def step(c):
            m, S, smin = c
            ffsv = plsc.all_reduce_ffs(m)
            row = g * LANES + ffsv
            vs = [plsc.load_gather(bufl, [row, j * LANES + lanes])
                  for j in range(8)]
            vs += [plsc.load_gather(bufr, [row, j * LANES + lanes])
                   for j in range(NG - 8)]
            S = _bmerge(S, _tournament(vs))
            smin = jnp.full((LANES,), jnp.min(S), jnp.float32)
            m = jnp.logical_and(m, lanes != ffsv)
            m = jnp.logical_and(m, rm >= smin)
            return m, S, smin

        _, S, smin = lax.while_loop(cond, step, (mask, S, smin))
        return S, smin

    S, _ = lax.fori_loop(0, NG, group, (s0, smin))
    return jnp.sum(S) * (1.0 / K)


def _sc_topk_body(flatl, flatr, rowmax, out,
                  bl0, bl1, br0, br1, rmb0, rmb1, res,
                  seml0, seml1, semr0, semr1, semm0, semm1):
    w = lax.axis_index("s") * NC + lax.axis_index("c")
    base = w * SPW
    bls, brs, rmbs = (bl0, bl1), (br0, br1), (rmb0, rmb1)
    semls, semrs, semms = (seml0, seml1), (semr0, semr1), (semm0, semm1)
    zero = jnp.zeros((LANES,), jnp.float32)
    lanes = lax.broadcasted_iota(jnp.int32, (LANES,), 0)

    def start(sl, b):
        pltpu.async_copy(flatl.at[pl.ds((base + sl) * FP, FP)], bls[b], semls[b])
        pltpu.async_copy(flatr.at[pl.ds((base + sl) * FP, FP)], brs[b], semrs[b])
        pltpu.async_copy(rowmax.at[base + sl], rmbs[b], semms[b])

    for b in (0, 1):
        start(b, b)

    def pair(i, carry):
        r0, r1 = carry
        for b in (0, 1):
            sl = i + b
            pltpu.make_async_copy(flatl.at[pl.ds(0, FP)], bls[b], semls[b]).wait()
            pltpu.make_async_copy(flatr.at[pl.ds(0, FP)], brs[b], semrs[b]).wait()
            pltpu.make_async_copy(rowmax.at[base], rmbs[b], semms[b]).wait()
            t = _process_slice(bls[b], brs[b], rmbs[b])

            @pl.when(sl + 2 < SPW)
            def _():
                start(sl + 2, b)

            r0 = jnp.where(lanes == sl, t, r0)
            r1 = jnp.where(lanes == sl - LANES, t, r1)
        return r0, r1

    r0, r1 = pl.loop(0, SPW, step=2, init_carry=(zero, zero))(pair)
    for j in range(OUTP // LANES):
        res[pl.ds(j * LANES, LANES)] = (r0, r1)[j] if j < 2 else zero
    pltpu.sync_copy(res.at[pl.ds(0, OUTP)], out.at[w])


_sc_topk = pl.kernel(
    _sc_topk_body,
    out_type=jax.ShapeDtypeStruct((NW, OUTP), jnp.float32),
    mesh=plsc.VectorSubcoreMesh(core_axis_name="c", subcore_axis_name="s",
                                num_cores=NC, num_subcores=NS),
    scratch_types=[
        pltpu.VMEM((FP, 128), jnp.float32),
        pltpu.VMEM((FP, 128), jnp.float32),
        pltpu.VMEM((FP, 128), jnp.float32),
        pltpu.VMEM((FP, 128), jnp.float32),
        pltpu.VMEM((T,), jnp.float32),
        pltpu.VMEM((T,), jnp.float32),
        pltpu.VMEM((OUTP,), jnp.float32),
        pltpu.SemaphoreType.DMA,
        pltpu.SemaphoreType.DMA,
        pltpu.SemaphoreType.DMA,
        pltpu.SemaphoreType.DMA,
        pltpu.SemaphoreType.DMA,
        pltpu.SemaphoreType.DMA,
    ],
    compiler_params=pltpu.CompilerParams(needs_layout_passes=False),
)


def kernel(x):
    B, C, _, _ = x.shape
    xr = x.reshape(B * C, F, T)
    outs = []
    for k in range(NCHUNK):
        xc = jax.lax.slice_in_dim(xr, k * CBC, (k + 1) * CBC, axis=0)
        pooledl, pooledr, rowmax = _tc_pool(xc)
        o = _sc_topk(pooledl, pooledr, rowmax.reshape(CBC, T))
        outs.append(o[:, :SPW].reshape(CBC))
    return jnp.concatenate(outs).reshape(B, C)


# speedup vs baseline: 1.4845x; 1.1617x over previous
import jax
import jax.numpy as jnp
from jax import lax
from jax.experimental import pallas as pl
from jax.experimental.pallas import tpu as pltpu
from jax.experimental.pallas import tpu_sc as plsc

W = 8
K = 16
F = 224
T = 224
FO = F - W + 1
BC = 4 * 192
NEG = float("-inf")
GCH = 8
FP = 224
HR = BC * FP

NC = 2
NS = 16
NW = NC * NS
NCHUNK = 1
CBC = BC // NCHUNK
SPW = CBC // NW
LANES = 16
NG = T // LANES
OUTP = 128


def _pool_body(x_ref, ol_ref, or_ref, rm_ref):
    a = x_ref[...]
    h = a
    for d in (1, 2, 4):
        h = h + jnp.concatenate([h[:, d:, :], h[:, :d, :]], axis=1)
    h = jnp.swapaxes(h, 1, 2)
    for d in (1, 2, 4):
        h = h + jnp.concatenate([h[:, d:, :], h[:, :d, :]], axis=1)
    row = lax.broadcasted_iota(jnp.int32, h.shape, 1)
    col = lax.broadcasted_iota(jnp.int32, h.shape, 2)
    valid = jnp.logical_and(row < FO, col < FO)
    o = jnp.where(valid, h * (1.0 / (W * W)), NEG)
    rm_ref[...] = jnp.max(o, axis=2)[:, None, :]
    o256 = jnp.concatenate(
        [o, jnp.full((GCH, FP, 256 - T), NEG, jnp.float32)], axis=2)
    ol_ref[...] = o256[:, :, 0:128].reshape(GCH * FP, 128)
    or_ref[...] = o256[:, :, 128:256].reshape(GCH * FP, 128)


def _tc_pool(xr):
    n = xr.shape[0]
    return pl.pallas_call(
        _pool_body,
        grid=(n // GCH,),
        in_specs=[pl.BlockSpec((GCH, F, T), lambda i: (i, 0, 0))],
        out_specs=[pl.BlockSpec((GCH * FP, 128), lambda i: (i, 0)),
                   pl.BlockSpec((GCH * FP, 128), lambda i: (i, 0)),
                   pl.BlockSpec((GCH, 1, T), lambda i: (i, 0, 0))],
        out_shape=[jax.ShapeDtypeStruct((n * FP, 128), jnp.float32),
                   jax.ShapeDtypeStruct((n * FP, 128), jnp.float32),
                   jax.ShapeDtypeStruct((n, 1, T), jnp.float32)],
    )(xr)


def _bmerge(a, b):
    return jnp.sort(jnp.maximum(a, jnp.flip(b)))


def _tournament(vs):
    level = [jnp.sort(v) for v in vs]
    while len(level) > 1:
        nxt = [_bmerge(level[i], level[i + 1])
               for i in range(0, len(level) - 1, 2)]
        if len(level) % 2:
            nxt.append(level[-1])
        level = nxt
    return level[0]


def _process_slice(bufl, bufr, rmb):
    lanes = lax.broadcasted_iota(jnp.int32, (LANES,), 0)
    srm = _tournament([rmb[pl.ds(g * LANES, LANES)] for g in range(NG)])
    smin = jnp.full((LANES,), jnp.min(srm), jnp.float32)
    s0 = jnp.full((LANES,), NEG, jnp.float32)

    def group(g, carry):
        S, smin = carry
        rm = rmb[pl.ds(g * LANES, LANES)]
        mask = rm >= smin

        def cond(c):
            m, _, _ = c
            return jnp.any(m)

        def step(c):
            m, S, smin = c
            ffsv = plsc.all_reduce_ffs(m)
            row = g * LANES + ffsv
            vs = [plsc.load_gather(bufl, [row, j * LANES + lanes])
                  for j in range(8)]
            vs += [plsc.load_gather(bufr, [row, j * LANES + lanes])
                   for j in range(NG - 8)]
            S = _bmerge(S, _tournament(vs))
            smin = jnp.full((LANES,), jnp.min(S), jnp.float32)
            m = jnp.logical_and(m, lanes != ffsv)
            m = jnp.logical_and(m, rm >= smin)
            return m, S, smin

        _, S, smin = lax.while_loop(cond, step, (mask, S, smin))
        return S, smin

    S, _ = lax.fori_loop(0, NG, group, (s0, smin))
    return jnp.sum(S) * (1.0 / K)


def _sc_topk_body(flatl, flatr, rowmax, out,
                  bl0, bl1, br0, br1, rmb0, rmb1, res,
                  seml0, seml1, semr0, semr1, semm0, semm1):
    w = lax.axis_index("s") * NC + lax.axis_index("c")
    base = w * SPW
    bls, brs, rmbs = (bl0, bl1), (br0, br1), (rmb0, rmb1)
    semls, semrs, semms = (seml0, seml1), (semr0, semr1), (semm0, semm1)
    zero = jnp.zeros((LANES,), jnp.float32)
    lanes = lax.broadcasted_iota(jnp.int32, (LANES,), 0)

    def start(sl, b):
        pltpu.async_copy(flatl.at[pl.ds((base + sl) * FP, FP)], bls[b], semls[b])
        pltpu.async_copy(flatr.at[pl.ds((base + sl) * FP, FP)], brs[b], semrs[b])
        pltpu.async_copy(rowmax.at[base + sl], rmbs[b], semms[b])

    for b in (0, 1):
        start(b, b)

    def pair(i, carry):
        r0, r1 = carry
        for b in (0, 1):
            sl = i + b
            pltpu.make_async_copy(flatl.at[pl.ds(0, FP)], bls[b], semls[b]).wait()
            pltpu.make_async_copy(flatr.at[pl.ds(0, FP)], brs[b], semrs[b]).wait()
            pltpu.make_async_copy(rowmax.at[base], rmbs[b], semms[b]).wait()
            t = _process_slice(bls[b], brs[b], rmbs[b])

            @pl.when(sl + 2 < SPW)
            def _():
                start(sl + 2, b)

            r0 = jnp.where(lanes == sl, t, r0)
            r1 = jnp.where(lanes == sl - LANES, t, r1)
        return r0, r1

    r0, r1 = pl.loop(0, SPW, step=2, init_carry=(zero, zero))(pair)
    for j in range(OUTP // LANES):
        res[pl.ds(j * LANES, LANES)] = (r0, r1)[j] if j < 2 else zero
    pltpu.sync_copy(res.at[pl.ds(0, OUTP)], out.at[w])


_sc_topk = pl.kernel(
    _sc_topk_body,
    out_type=jax.ShapeDtypeStruct((NW, OUTP), jnp.float32),
    mesh=plsc.VectorSubcoreMesh(core_axis_name="c", subcore_axis_name="s",
                                num_cores=NC, num_subcores=NS),
    scratch_types=[
        pltpu.VMEM((FP, 128), jnp.float32),
        pltpu.VMEM((FP, 128), jnp.float32),
        pltpu.VMEM((FP, 128), jnp.float32),
        pltpu.VMEM((FP, 128), jnp.float32),
        pltpu.VMEM((T,), jnp.float32),
        pltpu.VMEM((T,), jnp.float32),
        pltpu.VMEM((OUTP,), jnp.float32),
        pltpu.SemaphoreType.DMA,
        pltpu.SemaphoreType.DMA,
        pltpu.SemaphoreType.DMA,
        pltpu.SemaphoreType.DMA,
        pltpu.SemaphoreType.DMA,
        pltpu.SemaphoreType.DMA,
    ],
    compiler_params=pltpu.CompilerParams(needs_layout_passes=False),
)


def kernel(x):
    B, C, _, _ = x.shape
    xr = x.reshape(B * C, F, T)
    outs = []
    for k in range(NCHUNK):
        xc = jax.lax.slice_in_dim(xr, k * CBC, (k + 1) * CBC, axis=0)
        pooledl, pooledr, rowmax = _tc_pool(xc)
        o = _sc_topk(pooledl, pooledr, rowmax.reshape(CBC, T))
        outs.append(o[:, :SPW].reshape(CBC))
    return jnp.concatenate(outs).reshape(B, C)


# speedup vs baseline: 1.7427x; 1.1740x over previous
import jax
import jax.numpy as jnp
from jax import lax
from jax.experimental import pallas as pl
from jax.experimental.pallas import tpu as pltpu
from jax.experimental.pallas import tpu_sc as plsc

W = 8
K = 16
F = 224
T = 224
FO = F - W + 1
BC = 4 * 192
NEG = float("-inf")
GCH = 8
FP = 224
HR = BC * FP

NC = 2
NS = 16
NW = NC * NS
NCHUNK = 1
CBC = BC // NCHUNK
SPW = CBC // NW
LANES = 16
NG = T // LANES
OUTP = 128


def _pool_body(x_ref, ol_ref, or_ref, rm_ref):
    a = x_ref[...]
    h = a
    for d in (1, 2, 4):
        h = h + jnp.concatenate([h[:, d:, :], h[:, :d, :]], axis=1)
    h = jnp.swapaxes(h, 1, 2)
    for d in (1, 2, 4):
        h = h + jnp.concatenate([h[:, d:, :], h[:, :d, :]], axis=1)
    row = lax.broadcasted_iota(jnp.int32, h.shape, 1)
    col = lax.broadcasted_iota(jnp.int32, h.shape, 2)
    valid = jnp.logical_and(row < FO, col < FO)
    o = jnp.where(valid, h * (1.0 / (W * W)), NEG)
    rm_ref[...] = jnp.max(o, axis=2)[:, None, :]
    o256 = jnp.concatenate(
        [o, jnp.full((GCH, FP, 256 - T), NEG, jnp.float32)], axis=2)
    ol_ref[...] = o256[:, :, 0:128].reshape(GCH * FP, 128)
    or_ref[...] = o256[:, :, 128:256].reshape(GCH * FP, 128)


def _tc_pool(xr):
    n = xr.shape[0]
    return pl.pallas_call(
        _pool_body,
        grid=(n // GCH,),
        in_specs=[pl.BlockSpec((GCH, F, T), lambda i: (i, 0, 0))],
        out_specs=[pl.BlockSpec((GCH * FP, 128), lambda i: (i, 0)),
                   pl.BlockSpec((GCH * FP, 128), lambda i: (i, 0)),
                   pl.BlockSpec((GCH, 1, T), lambda i: (i, 0, 0))],
        out_shape=[jax.ShapeDtypeStruct((n * FP, 128), jnp.float32),
                   jax.ShapeDtypeStruct((n * FP, 128), jnp.float32),
                   jax.ShapeDtypeStruct((n, 1, T), jnp.float32)],
    )(xr)


def _bmerge(a, b):
    return jnp.sort(jnp.maximum(a, jnp.flip(b)))


def _tournament(vs):
    level = [jnp.sort(v) for v in vs]
    while len(level) > 1:
        nxt = [_bmerge(level[i], level[i + 1])
               for i in range(0, len(level) - 1, 2)]
        if len(level) % 2:
            nxt.append(level[-1])
        level = nxt
    return level[0]


def _kv_merge(ak, av, bk, bv):
    fbk, fbv = jnp.flip(bk), jnp.flip(bv)
    ck = jnp.maximum(ak, fbk)
    cv = jnp.where(ak >= fbk, av, fbv)
    return plsc.sort_key_val(ck, cv, descending=False)


def _kv_tournament(kvs):
    level = [plsc.sort_key_val(k, v, descending=False) for k, v in kvs]
    while len(level) > 1:
        nxt = [_kv_merge(*level[i], *level[i + 1])
               for i in range(0, len(level) - 1, 2)]
        if len(level) % 2:
            nxt.append(level[-1])
        level = nxt
    return level[0]


def _sc_topk_body(flatl, flatr, rowmax, out,
                  rmb0, rmb1, idx0, idx1, gl0, gl1, gr0, gr1, res,
                  semm0, semm1, segl0, segl1, segr0, segr1):
    w = lax.axis_index("s") * NC + lax.axis_index("c")
    base = w * SPW
    rmbs, idxs = (rmb0, rmb1), (idx0, idx1)
    gls, grs = (gl0, gl1), (gr0, gr1)
    semms, segls, segrs = (semm0, semm1), (segl0, segl1), (segr0, segr1)
    zero = jnp.zeros((LANES,), jnp.float32)
    lanes = lax.broadcasted_iota(jnp.int32, (LANES,), 0)
    s0 = jnp.full((LANES,), NEG, jnp.float32)

    def phase_a(sl, b):
        pltpu.make_async_copy(rowmax.at[base], rmbs[b], semms[b]).wait()
        kvs = [(rmbs[b][pl.ds(g * LANES, LANES)], g * LANES + lanes)
               for g in range(NG)]
        _, sv = _kv_tournament(kvs)
        idxs[b][...] = (base + sl) * FP + sv
        pltpu.async_copy(flatl.at[plsc.Indices(idxs[b])], gls[b], segls[b])
        pltpu.async_copy(flatr.at[plsc.Indices(idxs[b])], grs[b], segrs[b])

    def phase_b(b):
        pltpu.make_async_copy(flatl.at[plsc.Indices(idxs[b])],
                              gls[b], segls[b]).wait()
        pltpu.make_async_copy(flatr.at[plsc.Indices(idxs[b])],
                              grs[b], segrs[b]).wait()
        S = s0
        for k in range(LANES):
            vs = [gls[b][k, pl.ds(j * LANES, LANES)] for j in range(8)]
            vs += [grs[b][k, pl.ds(j * LANES, LANES)] for j in range(NG - 8)]
            S = _bmerge(S, _tournament(vs))
        return jnp.sum(S) * (1.0 / K)

    pltpu.async_copy(rowmax.at[base], rmbs[0], semms[0])
    pltpu.async_copy(rowmax.at[base + 1], rmbs[1], semms[1])
    phase_a(0, 0)
    pltpu.async_copy(rowmax.at[base + 2], rmbs[0], semms[0])

    def pair(i, carry):
        r0, r1 = carry
        for b in (0, 1):
            sl = i + b
            nb = 1 - b

            @pl.when(sl + 1 < SPW)
            def _():
                phase_a(sl + 1, nb)

                @pl.when(sl + 3 < SPW)
                def _():
                    pltpu.async_copy(rowmax.at[base + sl + 3],
                                     rmbs[nb], semms[nb])

            t = phase_b(b)
            r0 = jnp.where(lanes == sl, t, r0)
            r1 = jnp.where(lanes == sl - LANES, t, r1)
        return r0, r1

    r0, r1 = pl.loop(0, SPW, step=2, init_carry=(zero, zero))(pair)
    for j in range(OUTP // LANES):
        res[pl.ds(j * LANES, LANES)] = (r0, r1)[j] if j < 2 else zero
    pltpu.sync_copy(res.at[pl.ds(0, OUTP)], out.at[w])


_sc_topk = pl.kernel(
    _sc_topk_body,
    out_type=jax.ShapeDtypeStruct((NW, OUTP), jnp.float32),
    mesh=plsc.VectorSubcoreMesh(core_axis_name="c", subcore_axis_name="s",
                                num_cores=NC, num_subcores=NS),
    scratch_types=[
        pltpu.VMEM((T,), jnp.float32),
        pltpu.VMEM((T,), jnp.float32),
        pltpu.VMEM((LANES,), jnp.int32),
        pltpu.VMEM((LANES,), jnp.int32),
        pltpu.VMEM((LANES, 128), jnp.float32),
        pltpu.VMEM((LANES, 128), jnp.float32),
        pltpu.VMEM((LANES, 128), jnp.float32),
        pltpu.VMEM((LANES, 128), jnp.float32),
        pltpu.VMEM((OUTP,), jnp.float32),
        pltpu.SemaphoreType.DMA,
        pltpu.SemaphoreType.DMA,
        pltpu.SemaphoreType.DMA,
        pltpu.SemaphoreType.DMA,
        pltpu.SemaphoreType.DMA,
        pltpu.SemaphoreType.DMA,
    ],
    compiler_params=pltpu.CompilerParams(needs_layout_passes=False),
)


def kernel(x):
    B, C, _, _ = x.shape
    xr = x.reshape(B * C, F, T)
    outs = []
    for k in range(NCHUNK):
        xc = jax.lax.slice_in_dim(xr, k * CBC, (k + 1) * CBC, axis=0)
        pooledl, pooledr, rowmax = _tc_pool(xc)
        o = _sc_topk(pooledl, pooledr, rowmax.reshape(CBC, T))
        outs.append(o[:, :SPW].reshape(CBC))
    return jnp.concatenate(outs).reshape(B, C)
